# baseline (speedup 1.0000x reference)
import jax
import jax.numpy as jnp
from jax import lax
from jax.experimental import pallas as pl
from jax.experimental.pallas import tpu as pltpu
from jax.experimental.pallas import tpu_sc as plsc

N = 10000
E = 160000
D = 256
H = 4
DH = 64
G = 64
GH = 256

NB = 10
BN = N // NB

NSUB = 16
EPT = E // NSUB

NPAD = 10240
NPT = NPAD // NSUB
BN3 = NPAD // NB

C = 80
NCH = EPT // C


def _k0_body(x_ref, w_ref, av_ref, h_ref, tbl_ref):
    xb = x_ref[...]
    h = jnp.dot(xb, w_ref[...], preferred_element_type=jnp.float32)
    h_ref[...] = h.reshape(BN, 2, 128)
    tbl_ref[...] = jnp.dot(h, av_ref[...], preferred_element_type=jnp.float32)


SCE = 2000
CPS = SCE // C


def _ksc_body(tbl_hbm, h_hbm, src_hbm, dst_hbm, b_hbm, xatt_hbm,
              esrcL, edstL,
              gsrc0, gsrc1, ed0, ed1,
              ib00, ib10, ib20, ib30, ib01, ib11, ib21, ib31,
              av00, av10, av20, av30, av01, av11, av21, av31,
              w00, w10, w01, w11, rows0, rows1, sb0, sb1, bbuf,
              tbl_sh, acc_sh, s0_sh, s1_sh,
              gsem0, gsem1, lsem0, lsem1, ssem0, ssem1):
    c = lax.axis_index("c")
    t = lax.axis_index("s")
    zero = jnp.zeros((16,), jnp.float32)

    gsrc = (gsrc0, gsrc1)
    ed = (ed0, ed1)
    ib = ((ib00, ib10, ib20, ib30), (ib01, ib11, ib21, ib31))
    av = ((av00, av10, av20, av30), (av01, av11, av21, av31))
    wv = ((w00, w10), (w01, w11))
    rows = (rows0, rows1)
    gsem = (gsem0, gsem1)
    lsem = (lsem0, lsem1)
    ssem = (ssem0, ssem1)

    @pl.when(t == 0)
    def _():
        pltpu.sync_copy(tbl_hbm, tbl_sh)
    pltpu.sync_copy(b_hbm.at[pl.ds(128 * c, 128)], bbuf)

    def zrow(r, _):
        for j in range(8):
            rows0[r, pl.ds(j * 16, 16)] = zero
        return _
    lax.fori_loop(0, C, zrow, None)
    for g in range(5):
        w00[pl.ds(g * 16, 16)] = zero
    def zcp(k, _):
        pltpu.sync_copy(rows0, acc_sh.at[pl.ds(t * NPT + k * C, C)])
        pltpu.sync_copy(w00, s0_sh.at[pl.ds(t * NPT + k * C, C)])
        pltpu.sync_copy(w00, s1_sh.at[pl.ds(t * NPT + k * C, C)])
        return _
    lax.fori_loop(0, NPT // C, zcp, None)
    plsc.subcore_barrier()

    def stage_a(j, p, first):
        @pl.when(j < NCH)
        def _():
            @pl.when(j % CPS == 0)
            def _():
                sbase = t * EPT + (j // CPS) * SCE
                pltpu.sync_copy(src_hbm.at[pl.ds(sbase, SCE)], esrcL)
                pltpu.sync_copy(dst_hbm.at[pl.ds(sbase, SCE)], edstL)

            if not first:
                pltpu.make_async_copy(
                    rows[p], acc_sh.at[ed[p]], ssem[p]).wait()
                pltpu.make_async_copy(
                    wv[p][0], s0_sh.at[ed[p]], ssem[p]).wait()
                pltpu.make_async_copy(
                    wv[p][1], s1_sh.at[ed[p]], ssem[p]).wait()

            off = (j % CPS) * C

            def fix(k, _=None):
                sl = pl.ds(k * 16, 16)
                ev = esrcL[pl.ds(off + k * 16, 16)]
                dv = edstL[pl.ds(off + k * 16, 16)]
                gsrc[p][sl] = ev * 2 + c
                ed[p][sl] = dv
                ib[p][0][sl] = ev * 8 + 2 * c
                ib[p][1][sl] = ev * 8 + (2 * c + 1)
                ib[p][2][sl] = dv * 8 + (4 + 2 * c)
                ib[p][3][sl] = dv * 8 + (5 + 2 * c)
                return _
            lax.fori_loop(0, C // 16, fix, None)

            pltpu.async_copy(h_hbm.at[gsrc[p]], rows[p], gsem[p])
            for q in range(4):
                pltpu.async_copy(tbl_sh.at[ib[p][q]], av[p][q], lsem[p])

    def stage_b(p):
        for q in range(4):
            pltpu.make_async_copy(tbl_sh.at[ib[p][q]], av[p][q],
                                  lsem[p]).wait()

        def wgroup(g, _):
            sl = pl.ds(g * 16, 16)
            for j in range(2):
                e = (av[p][0][sl] + av[p][2][sl]) if j == 0 \
                    else (av[p][1][sl] + av[p][3][sl])
                e = jnp.where(e > 0.0, e, 0.2 * e)
                wv[p][j][sl] = jnp.exp(e)
            return _
        lax.fori_loop(0, C // 16, wgroup, None)

        pltpu.make_async_copy(h_hbm.at[gsrc[p]], rows[p], gsem[p]).wait()

        @plsc.parallel_loop(0, C // 16, unroll=5)
        def mgroup(g):
            wv0 = wv[p][0][pl.ds(g * 16, 16)]
            wv1 = wv[p][1][pl.ds(g * 16, 16)]
            for k in range(16):
                r = g * 16 + k
                f0 = jnp.full((16,), wv0[k])
                f1 = jnp.full((16,), wv1[k])
                for j in range(8):
                    sl = pl.ds(j * 16, 16)
                    rows[p][r, sl] = rows[p][r, sl] * (f0 if j < 4 else f1)

        pltpu.async_copy(rows[p], acc_sh.at[ed[p]], ssem[p], add=True)
        pltpu.async_copy(wv[p][0], s0_sh.at[ed[p]], ssem[p], add=True)
        pltpu.async_copy(wv[p][1], s1_sh.at[ed[p]], ssem[p], add=True)

    stage_a(jnp.int32(0), 0, True)
    stage_a(jnp.int32(1), 1, True)

    def pipe(i, _):
        j = i * 2
        stage_b(0)
        stage_a(j + 2, 0, False)
        stage_b(1)
        stage_a(j + 3, 1, False)
        return _
    lax.fori_loop(0, NCH // 2, pipe, None)
    stage_b(0)

    for p in (0, 1):
        pltpu.make_async_copy(rows[p], acc_sh.at[ed[p]], ssem[p]).wait()
        pltpu.make_async_copy(wv[p][0], s0_sh.at[ed[p]], ssem[p]).wait()
        pltpu.make_async_copy(wv[p][1], s1_sh.at[ed[p]], ssem[p]).wait()

    plsc.subcore_barrier()

    def divchunk(k, _carry):
        nbase = t * NPT + k * C

        @pl.when(nbase < N)
        def _():
            pltpu.sync_copy(s0_sh.at[pl.ds(nbase, C)], sb0)
            pltpu.sync_copy(s1_sh.at[pl.ds(nbase, C)], sb1)
            pltpu.sync_copy(acc_sh.at[pl.ds(nbase, C)], rows0)

            def dgroup(g, _=None):
                sv0 = sb0[pl.ds(g * 16, 16)]
                sv1 = sb1[pl.ds(g * 16, 16)]
                for k2 in range(16):
                    r = g * 16 + k2
                    r0 = 1.0 / (jnp.full((16,), sv0[k2]) + 1e-16)
                    r1 = 1.0 / (jnp.full((16,), sv1[k2]) + 1e-16)
                    for j in range(8):
                        sl = pl.ds(j * 16, 16)
                        rows0[r, sl] = (rows0[r, sl] * (r0 if j < 4 else r1)
                                        + bbuf[sl])
                return _
            lax.fori_loop(0, C // 16, dgroup, None)
            pltpu.sync_copy(rows0,
                            xatt_hbm.at[pl.ds(nbase, C), pl.ds(128 * c, 128)])
        return _carry
    lax.fori_loop(0, NPT // C, divchunk, None)


def _k3_body(xatt_ref, batch_ref, xg_ref, wg_ref, bg_ref, wf_ref, bf_ref,
             w1_ref, b1_ref, w2_ref, b2_ref, out_ref, pool_acc):
    i = pl.program_id(0)

    @pl.when(i == 0)
    def _():
        pool_acc[...] = jnp.zeros((G, D), jnp.float32)

    xb = xatt_ref[...]
    gate = jax.nn.sigmoid(
        jnp.dot(xb, wg_ref[...], preferred_element_type=jnp.float32)
        + bg_ref[...])
    feat = (jnp.dot(xb, wf_ref[...], preferred_element_type=jnp.float32)
            + bf_ref[...])
    contrib = gate * feat
    bb = batch_ref[0, 0, :]
    oh = (bb[None, :] == lax.broadcasted_iota(jnp.int32, (G, BN), 0)
          ).astype(jnp.float32)
    pool_acc[...] += jnp.dot(oh, contrib, preferred_element_type=jnp.float32)

    @pl.when(i == NB - 1)
    def _():
        xc = jnp.concatenate([xg_ref[...], pool_acc[...]], axis=1)
        y = (jnp.dot(xc, w1_ref[...], preferred_element_type=jnp.float32)
             + b1_ref[...])
        y = y * jax.nn.sigmoid(y)
        out_ref[...] = (
            jnp.dot(y, w2_ref[...], preferred_element_type=jnp.float32)
            + b2_ref[...])


@jax.jit
def kernel(x, edge_index, batch, x_global, W_gat, a_src, a_dst, b_gat,
           Wr_gate, br_gate, Wr_feat, br_feat, W1, b1, W2, b2):
    src = edge_index[0]
    dst = edge_index[1]
    eye = jnp.eye(H, dtype=jnp.float32)
    A_src = (a_src[:, :, None] * eye[:, None, :]).reshape(D, H)
    A_dst = (a_dst[:, :, None] * eye[:, None, :]).reshape(D, H)
    A_all = jnp.concatenate([A_src, A_dst], axis=1)

    h3, tbl = pl.pallas_call(
        _k0_body,
        grid=(NB,),
        in_specs=[
            pl.BlockSpec((BN, D), lambda i: (i, 0)),
            pl.BlockSpec((D, D), lambda i: (0, 0)),
            pl.BlockSpec((D, 2 * H), lambda i: (0, 0)),
        ],
        out_specs=[
            pl.BlockSpec((BN, 2, 128), lambda i: (i, 0, 0)),
            pl.BlockSpec((BN, 2 * H), lambda i: (i, 0)),
        ],
        out_shape=[
            jax.ShapeDtypeStruct((N, 2, 128), jnp.float32),
            jax.ShapeDtypeStruct((N, 2 * H), jnp.float32),
        ],
    )(x, W_gat, A_all)

    h_flat = h3.reshape(2 * N, 128)
    tbl_flat = tbl.reshape(2 * H * N)

    ksc = pl.kernel(
        _ksc_body,
        out_type=jax.ShapeDtypeStruct((N, D), jnp.float32),
        mesh=plsc.VectorSubcoreMesh(core_axis_name="c", subcore_axis_name="s"),
        compiler_params=pltpu.CompilerParams(needs_layout_passes=False),
        scratch_types=(
            [pltpu.VMEM((SCE,), jnp.int32)] * 2
            + [pltpu.VMEM((C,), jnp.int32)] * 4
            + [pltpu.VMEM((C,), jnp.int32)] * 8
            + [pltpu.VMEM((C,), jnp.float32)] * 8
            + [pltpu.VMEM((C,), jnp.float32)] * 4
            + [pltpu.VMEM((C, 128), jnp.float32)] * 2
            + [pltpu.VMEM((C,), jnp.float32)] * 2
            + [pltpu.VMEM((128,), jnp.float32)]
            + [pltpu.VMEM_SHARED((2 * H * N,), jnp.float32)]
            + [pltpu.VMEM_SHARED((NPAD, 128), jnp.float32)]
            + [pltpu.VMEM_SHARED((NPAD,), jnp.float32)] * 2
            + [pltpu.SemaphoreType.DMA] * 6
        ),
    )
    x_att = ksc(tbl_flat, h_flat, src, dst, b_gat)

    batch_r = batch.reshape(NB, 1, BN)

    x_local = pl.pallas_call(
        _k3_body,
        grid=(NB,),
        in_specs=[
            pl.BlockSpec((BN, D), lambda i: (i, 0)),
            pl.BlockSpec((1, 1, BN), lambda i: (i, 0, 0)),
            pl.BlockSpec((G, GH), lambda i: (0, 0)),
            pl.BlockSpec((D, D), lambda i: (0, 0)),
            pl.BlockSpec((1, D), lambda i: (0, 0)),
            pl.BlockSpec((D, D), lambda i: (0, 0)),
            pl.BlockSpec((1, D), lambda i: (0, 0)),
            pl.BlockSpec((D + GH, D), lambda i: (0, 0)),
            pl.BlockSpec((1, D), lambda i: (0, 0)),
            pl.BlockSpec((D, D), lambda i: (0, 0)),
            pl.BlockSpec((1, D), lambda i: (0, 0)),
        ],
        out_specs=pl.BlockSpec((G, D), lambda i: (0, 0)),
        out_shape=jax.ShapeDtypeStruct((G, D), jnp.float32),
        scratch_shapes=[pltpu.VMEM((G, D), jnp.float32)],
    )(x_att, batch_r, x_global, Wr_gate,
      br_gate.reshape(1, D), Wr_feat, br_feat.reshape(1, D),
      W1, b1.reshape(1, D), W2, b2.reshape(1, D))

    return (x_att, x_local)


# speedup vs baseline: 1.3365x; 1.3365x over previous
import jax
import jax.numpy as jnp
from jax import lax
from jax.experimental import pallas as pl
from jax.experimental.pallas import tpu as pltpu
from jax.experimental.pallas import tpu_sc as plsc

N = 10000
E = 160000
D = 256
H = 4
DH = 64
G = 64
GH = 256

NB = 10
BN = N // NB

NSUB = 16
EPT = E // NSUB

NPAD = 10240
NPT = NPAD // NSUB
BN3 = NPAD // NB

C = 80
NCH = EPT // C


def _k0_body(x_ref, w_ref, av_ref, h_ref, tbl_ref):
    xb = x_ref[...]
    h = jnp.dot(xb, w_ref[...], preferred_element_type=jnp.float32)
    h_ref[...] = h.reshape(BN, 2, 128)
    tbl_ref[...] = jnp.dot(h, av_ref[...], preferred_element_type=jnp.float32)


SCE = 2000
CPS = SCE // C


def _ksc_body(tbl_hbm, h_hbm, src_hbm, dst_hbm, b_hbm, xatt_hbm,
              esrcL, edstL,
              gsrc0, gsrc1, ed0, ed1,
              ib00, ib10, ib20, ib30, ib01, ib11, ib21, ib31,
              av00, av10, av20, av30, av01, av11, av21, av31,
              w00, w10, w01, w11, rows0, rows1, sb0, sb1, bbuf,
              tbl_sh, acc_sh, s0_sh, s1_sh,
              gsem0, gsem1, lsem0, lsem1, ssem0, ssem1):
    c = lax.axis_index("c")
    t = lax.axis_index("s")
    zero = jnp.zeros((16,), jnp.float32)

    gsrc = (gsrc0, gsrc1)
    ed = (ed0, ed1)
    ib = ((ib00, ib10, ib20, ib30), (ib01, ib11, ib21, ib31))
    av = ((av00, av10, av20, av30), (av01, av11, av21, av31))
    wv = ((w00, w10), (w01, w11))
    rows = (rows0, rows1)
    gsem = (gsem0, gsem1)
    lsem = (lsem0, lsem1)
    ssem = (ssem0, ssem1)

    @pl.when(t == 0)
    def _():
        pltpu.sync_copy(tbl_hbm, tbl_sh)
    pltpu.sync_copy(b_hbm.at[pl.ds(128 * c, 128)], bbuf)

    def zrow(r, _):
        for j in range(8):
            rows0[r, pl.ds(j * 16, 16)] = zero
        return _
    lax.fori_loop(0, C, zrow, None)
    for g in range(5):
        w00[pl.ds(g * 16, 16)] = zero
    def zcp(k, _):
        pltpu.sync_copy(rows0, acc_sh.at[pl.ds(t * NPT + k * C, C)])
        pltpu.sync_copy(w00, s0_sh.at[pl.ds(t * NPT + k * C, C)])
        pltpu.sync_copy(w00, s1_sh.at[pl.ds(t * NPT + k * C, C)])
        return _
    lax.fori_loop(0, NPT // C, zcp, None)
    plsc.subcore_barrier()

    def stage_a(j, p, first):
        @pl.when(j < NCH)
        def _():
            @pl.when(j % CPS == 0)
            def _():
                sbase = t * EPT + (j // CPS) * SCE
                pltpu.sync_copy(src_hbm.at[pl.ds(sbase, SCE)], esrcL)
                pltpu.sync_copy(dst_hbm.at[pl.ds(sbase, SCE)], edstL)

            if not first:
                pltpu.make_async_copy(
                    rows[p], acc_sh.at[ed[p]], ssem[p]).wait()
                pltpu.make_async_copy(
                    wv[p][0], s0_sh.at[ed[p]], ssem[p]).wait()
                pltpu.make_async_copy(
                    wv[p][1], s1_sh.at[ed[p]], ssem[p]).wait()

            off = (j % CPS) * C

            def fix(k, _=None):
                sl = pl.ds(k * 16, 16)
                ev = esrcL[pl.ds(off + k * 16, 16)]
                dv = edstL[pl.ds(off + k * 16, 16)]
                gsrc[p][sl] = ev * 2 + c
                ed[p][sl] = dv
                ib[p][0][sl] = ev * 8 + 2 * c
                ib[p][1][sl] = ev * 8 + (2 * c + 1)
                ib[p][2][sl] = dv * 8 + (4 + 2 * c)
                ib[p][3][sl] = dv * 8 + (5 + 2 * c)
                return _
            lax.fori_loop(0, C // 16, fix, None)

            pltpu.async_copy(h_hbm.at[gsrc[p]], rows[p], gsem[p])
            for q in range(4):
                pltpu.async_copy(tbl_sh.at[ib[p][q]], av[p][q], lsem[p])

    def stage_b(p):
        for q in range(4):
            pltpu.make_async_copy(tbl_sh.at[ib[p][q]], av[p][q],
                                  lsem[p]).wait()

        def wgroup(g, _):
            sl = pl.ds(g * 16, 16)
            for j in range(2):
                e = (av[p][0][sl] + av[p][2][sl]) if j == 0 \
                    else (av[p][1][sl] + av[p][3][sl])
                e = jnp.where(e > 0.0, e, 0.2 * e)
                wv[p][j][sl] = jnp.exp(e)
            return _
        lax.fori_loop(0, C // 16, wgroup, None)

        pltpu.make_async_copy(h_hbm.at[gsrc[p]], rows[p], gsem[p]).wait()

        @plsc.parallel_loop(0, C // 16)
        def mgroup(g):
            wv0 = wv[p][0][pl.ds(g * 16, 16)]
            wv1 = wv[p][1][pl.ds(g * 16, 16)]
            for k in range(16):
                r = g * 16 + k
                f0 = jnp.full((16,), wv0[k])
                f1 = jnp.full((16,), wv1[k])
                for j in range(8):
                    sl = pl.ds(j * 16, 16)
                    rows[p][r, sl] = rows[p][r, sl] * (f0 if j < 4 else f1)

        pltpu.async_copy(rows[p], acc_sh.at[ed[p]], ssem[p], add=True)
        pltpu.async_copy(wv[p][0], s0_sh.at[ed[p]], ssem[p], add=True)
        pltpu.async_copy(wv[p][1], s1_sh.at[ed[p]], ssem[p], add=True)

    stage_a(jnp.int32(0), 0, True)
    stage_a(jnp.int32(1), 1, True)

    def pipe(i, _):
        j = i * 2
        stage_b(0)
        stage_a(j + 2, 0, False)
        stage_b(1)
        stage_a(j + 3, 1, False)
        return _
    lax.fori_loop(0, NCH // 2, pipe, None)
    stage_b(0)

    for p in (0, 1):
        pltpu.make_async_copy(rows[p], acc_sh.at[ed[p]], ssem[p]).wait()
        pltpu.make_async_copy(wv[p][0], s0_sh.at[ed[p]], ssem[p]).wait()
        pltpu.make_async_copy(wv[p][1], s1_sh.at[ed[p]], ssem[p]).wait()

    plsc.subcore_barrier()

    def divchunk(k, _carry):
        nbase = t * NPT + k * C

        @pl.when(nbase < N)
        def _():
            pltpu.sync_copy(s0_sh.at[pl.ds(nbase, C)], sb0)
            pltpu.sync_copy(s1_sh.at[pl.ds(nbase, C)], sb1)
            pltpu.sync_copy(acc_sh.at[pl.ds(nbase, C)], rows0)

            def dgroup(g, _=None):
                sv0 = sb0[pl.ds(g * 16, 16)]
                sv1 = sb1[pl.ds(g * 16, 16)]
                for k2 in range(16):
                    r = g * 16 + k2
                    r0 = 1.0 / (jnp.full((16,), sv0[k2]) + 1e-16)
                    r1 = 1.0 / (jnp.full((16,), sv1[k2]) + 1e-16)
                    for j in range(8):
                        sl = pl.ds(j * 16, 16)
                        rows0[r, sl] = (rows0[r, sl] * (r0 if j < 4 else r1)
                                        + bbuf[sl])
                return _
            lax.fori_loop(0, C // 16, dgroup, None)
            pltpu.sync_copy(rows0,
                            xatt_hbm.at[pl.ds(nbase, C), pl.ds(128 * c, 128)])
        return _carry
    lax.fori_loop(0, NPT // C, divchunk, None)


def _k3_body(xatt_ref, batch_ref, xg_ref, wg_ref, bg_ref, wf_ref, bf_ref,
             w1_ref, b1_ref, w2_ref, b2_ref, out_ref, pool_acc):
    i = pl.program_id(0)

    @pl.when(i == 0)
    def _():
        pool_acc[...] = jnp.zeros((G, D), jnp.float32)

    xb = xatt_ref[...]
    gate = jax.nn.sigmoid(
        jnp.dot(xb, wg_ref[...], preferred_element_type=jnp.float32)
        + bg_ref[...])
    feat = (jnp.dot(xb, wf_ref[...], preferred_element_type=jnp.float32)
            + bf_ref[...])
    contrib = gate * feat
    bb = batch_ref[0, 0, :]
    oh = (bb[None, :] == lax.broadcasted_iota(jnp.int32, (G, BN), 0)
          ).astype(jnp.float32)
    pool_acc[...] += jnp.dot(oh, contrib, preferred_element_type=jnp.float32)

    @pl.when(i == NB - 1)
    def _():
        xc = jnp.concatenate([xg_ref[...], pool_acc[...]], axis=1)
        y = (jnp.dot(xc, w1_ref[...], preferred_element_type=jnp.float32)
             + b1_ref[...])
        y = y * jax.nn.sigmoid(y)
        out_ref[...] = (
            jnp.dot(y, w2_ref[...], preferred_element_type=jnp.float32)
            + b2_ref[...])


@jax.jit
def kernel(x, edge_index, batch, x_global, W_gat, a_src, a_dst, b_gat,
           Wr_gate, br_gate, Wr_feat, br_feat, W1, b1, W2, b2):
    src = edge_index[0]
    dst = edge_index[1]
    eye = jnp.eye(H, dtype=jnp.float32)
    A_src = (a_src[:, :, None] * eye[:, None, :]).reshape(D, H)
    A_dst = (a_dst[:, :, None] * eye[:, None, :]).reshape(D, H)
    A_all = jnp.concatenate([A_src, A_dst], axis=1)

    h3, tbl = pl.pallas_call(
        _k0_body,
        grid=(NB,),
        in_specs=[
            pl.BlockSpec((BN, D), lambda i: (i, 0)),
            pl.BlockSpec((D, D), lambda i: (0, 0)),
            pl.BlockSpec((D, 2 * H), lambda i: (0, 0)),
        ],
        out_specs=[
            pl.BlockSpec((BN, 2, 128), lambda i: (i, 0, 0)),
            pl.BlockSpec((BN, 2 * H), lambda i: (i, 0)),
        ],
        out_shape=[
            jax.ShapeDtypeStruct((N, 2, 128), jnp.float32),
            jax.ShapeDtypeStruct((N, 2 * H), jnp.float32),
        ],
    )(x, W_gat, A_all)

    h_flat = h3.reshape(2 * N, 128)
    tbl_flat = tbl.reshape(2 * H * N)

    ksc = pl.kernel(
        _ksc_body,
        out_type=jax.ShapeDtypeStruct((N, D), jnp.float32),
        mesh=plsc.VectorSubcoreMesh(core_axis_name="c", subcore_axis_name="s"),
        compiler_params=pltpu.CompilerParams(needs_layout_passes=False),
        scratch_types=(
            [pltpu.VMEM((SCE,), jnp.int32)] * 2
            + [pltpu.VMEM((C,), jnp.int32)] * 4
            + [pltpu.VMEM((C,), jnp.int32)] * 8
            + [pltpu.VMEM((C,), jnp.float32)] * 8
            + [pltpu.VMEM((C,), jnp.float32)] * 4
            + [pltpu.VMEM((C, 128), jnp.float32)] * 2
            + [pltpu.VMEM((C,), jnp.float32)] * 2
            + [pltpu.VMEM((128,), jnp.float32)]
            + [pltpu.VMEM_SHARED((2 * H * N,), jnp.float32)]
            + [pltpu.VMEM_SHARED((NPAD, 128), jnp.float32)]
            + [pltpu.VMEM_SHARED((NPAD,), jnp.float32)] * 2
            + [pltpu.SemaphoreType.DMA] * 6
        ),
    )
    x_att = ksc(tbl_flat, h_flat, src, dst, b_gat)

    batch_r = batch.reshape(NB, 1, BN)

    x_local = pl.pallas_call(
        _k3_body,
        grid=(NB,),
        in_specs=[
            pl.BlockSpec((BN, D), lambda i: (i, 0)),
            pl.BlockSpec((1, 1, BN), lambda i: (i, 0, 0)),
            pl.BlockSpec((G, GH), lambda i: (0, 0)),
            pl.BlockSpec((D, D), lambda i: (0, 0)),
            pl.BlockSpec((1, D), lambda i: (0, 0)),
            pl.BlockSpec((D, D), lambda i: (0, 0)),
            pl.BlockSpec((1, D), lambda i: (0, 0)),
            pl.BlockSpec((D + GH, D), lambda i: (0, 0)),
            pl.BlockSpec((1, D), lambda i: (0, 0)),
            pl.BlockSpec((D, D), lambda i: (0, 0)),
            pl.BlockSpec((1, D), lambda i: (0, 0)),
        ],
        out_specs=pl.BlockSpec((G, D), lambda i: (0, 0)),
        out_shape=jax.ShapeDtypeStruct((G, D), jnp.float32),
        scratch_shapes=[pltpu.VMEM((G, D), jnp.float32)],
    )(x_att, batch_r, x_global, Wr_gate,
      br_gate.reshape(1, D), Wr_feat, br_feat.reshape(1, D),
      W1, b1.reshape(1, D), W2, b2.reshape(1, D))

    return (x_att, x_local)
